# baseline (speedup 1.0000x reference)
import functools

import jax
import jax.numpy as jnp
from jax import lax
from jax.experimental import pallas as pl
from jax.experimental.pallas import tpu as pltpu
from jax.experimental.pallas import tpu_sc as plsc

B = 16384
D = 32
L = 16

_info = plsc.get_sparse_core_info()
NC, NS = _info.num_cores, _info.num_subcores
NW = NC * NS
ROWS_PER_W = B // NW
SUB = 128
SUBS_PER_W = ROWS_PER_W // SUB
CHUNK_SUBS = 2
N_CHUNKS = SUBS_PER_W // CHUNK_SUBS
NT = 12


def _sc_body(eu, ei, eru, eri, idx_hbm, diff_out, reg_out,
             idx_v, rows_v, diff_v, vec_v, sem):
    wid = lax.axis_index("s") * NC + lax.axis_index("c")
    tables = [eu, eu, eu, ei, ei, ei, eru, eru, eru, eri, eri, eri]

    acc_sq = jnp.zeros((L,), jnp.float32)
    for g in range(N_CHUNKS):
        sub0 = wid * SUBS_PER_W + g * CHUNK_SUBS
        for t in range(NT):
            pltpu.sync_copy(idx_hbm.at[t, pl.ds(sub0, CHUNK_SUBS)],
                            idx_v.at[t])
        copies = []
        for t in range(NT):
            for j in range(CHUNK_SUBS):
                copies.append(pltpu.async_copy(
                    tables[t].at[idx_v.at[t, j]], rows_v.at[t, j], sem))
        for c in copies:
            c.wait()

        for j in range(CHUNK_SUBS):
            out_base = g * CHUNK_SUBS * SUB + j * SUB

            def body(r, acc, j=j, out_base=out_base):
                lo = [rows_v[t, j, r, pl.ds(0, L)] for t in range(NT)]
                hi = [rows_v[t, j, r, pl.ds(L, L)] for t in range(NT)]
                b_lo = lo[0] + lo[3] + lo[6] + lo[9]
                b_hi = hi[0] + hi[3] + hi[6] + hi[9]
                p_lo = lo[1] + lo[4] + lo[7] + lo[10]
                p_hi = hi[1] + hi[4] + hi[7] + hi[10]
                n_lo = lo[2] + lo[5] + lo[8] + lo[11]
                n_hi = hi[2] + hi[5] + hi[8] + hi[11]
                dv = b_lo * (n_lo - p_lo) + b_hi * (n_hi - p_hi)
                diff_v[out_base + r] = dv
                sq = acc
                for v in lo:
                    sq = sq + v * v
                for v in hi:
                    sq = sq + v * v
                return sq

            acc_sq = lax.fori_loop(0, SUB, body, acc_sq)

    vec_v[...] = acc_sq
    pltpu.sync_copy(diff_v, diff_out.at[pl.ds(wid * ROWS_PER_W, ROWS_PER_W)])
    pltpu.sync_copy(vec_v, reg_out.at[pl.ds(wid * L, L)])


def _finalize_body(diff_ref, reg_ref, loss_ref, regloss_ref):
    x = jnp.sum(diff_ref[...], axis=-1)
    sp = jnp.maximum(x, 0.0) + jnp.log1p(jnp.exp(-jnp.abs(x)))
    loss_ref[0, 0] = jnp.sum(sp) / float(B)
    regloss_ref[0, 0] = 0.5 * jnp.sum(reg_ref[...]) / float(B)


def kernel(user, item, user_pos, item_pos, user_neg, item_neg,
           rel_u, pos_rel_u, neg_rel_u, rel_i, pos_rel_i, neg_rel_i,
           embed_user, embed_item, embed_rel_u, embed_rel_i):
    idx_all = jnp.stack([user, user_pos, user_neg,
                         item, item_pos, item_neg,
                         rel_u, pos_rel_u, neg_rel_u,
                         rel_i, pos_rel_i, neg_rel_i]).astype(jnp.int32)
    idx_all = idx_all.reshape(NT, B // SUB, SUB)

    sc = pl.kernel(
        _sc_body,
        mesh=plsc.VectorSubcoreMesh(core_axis_name="c", subcore_axis_name="s"),
        compiler_params=pltpu.CompilerParams(use_tc_tiling_on_sc=False),
        out_type=[jax.ShapeDtypeStruct((B, L), jnp.float32),
                  jax.ShapeDtypeStruct((NW * L,), jnp.float32)],
        scratch_types=[
            pltpu.VMEM((NT, CHUNK_SUBS, SUB), jnp.int32),
            pltpu.VMEM((NT, CHUNK_SUBS, SUB, D), jnp.float32),
            pltpu.VMEM((ROWS_PER_W, L), jnp.float32),
            pltpu.VMEM((L,), jnp.float32),
            pltpu.SemaphoreType.DMA,
        ],
    )
    diff, reg_part = sc(embed_user, embed_item, embed_rel_u, embed_rel_i,
                        idx_all)

    loss, reg_loss = pl.pallas_call(
        _finalize_body,
        out_shape=[jax.ShapeDtypeStruct((1, 1), jnp.float32),
                   jax.ShapeDtypeStruct((1, 1), jnp.float32)],
        out_specs=[pl.BlockSpec(memory_space=pltpu.SMEM),
                   pl.BlockSpec(memory_space=pltpu.SMEM)],
    )(diff, reg_part.reshape(NW * L // 128, 128))
    return (loss[0, 0], reg_loss[0, 0])
